# baseline (speedup 1.0000x reference)
import jax
import jax.numpy as jnp
from jax import lax
from jax.experimental import pallas as pl
from jax.experimental.pallas import tpu as pltpu
from jax.experimental.pallas import tpu_sc as plsc

_PLANE_RES = 1024
_GRID_RES = 256
_FEAT = 4
_NC = 2
_NS = 16
_NW = _NC * _NS
_L = 16
_CHUNK = 128


def _floorfrac(v, res):
    p = v * jnp.float32(res - 1) + jnp.float32(0.5)
    i = p.astype(jnp.int32)
    f = p - i.astype(jnp.float32)
    ic = jnp.minimum(i, res - 2)
    return ic, f


class _Set:
    def __init__(self, s):
        (self.x, self.y, self.z, self.fr, self.hb) = s[0:5]
        self.pidx = s[5:8]
        self.gidx = s[8:16]
        (self.prow, self.grow, self.xsem, self.gsem) = s[16:20]


_SET_LEN = 20


def _set_types():
    return (
        [pltpu.VMEM((_CHUNK,), jnp.float32)] * 3
        + [pltpu.VMEM((6 * _CHUNK,), jnp.float32)]
        + [pltpu.VMEM((_CHUNK,), jnp.int32)]
        + [pltpu.VMEM((_CHUNK,), jnp.int32)] * 11
        + [pltpu.VMEM((3, _CHUNK, 4 * _FEAT), jnp.float32),
           pltpu.VMEM((8, _CHUNK, 2 * _FEAT), jnp.float32),
           pltpu.SemaphoreType.DMA,
           pltpu.SemaphoreType.DMA]
    )


_NBUF = 4


def _body(x_hbm, y_hbm, z_hbm, plane_hbm, grid_hbm, out_hbm, *s):
    sets = [_Set(s[k * _SET_LEN:(k + 1) * _SET_LEN]) for k in range(_NBUF)]
    out0, osem0, out1, osem1 = s[_NBUF * _SET_LEN:_NBUF * _SET_LEN + 4]
    outs = (out0, out1)
    osems = (osem0, osem1)

    wid = lax.axis_index("s") * _NC + lax.axis_index("c")
    npts = x_hbm.shape[0] // _NW
    nchunk = npts // _CHUNK
    last = nchunk - 1
    base = wid * npts

    lane = lax.iota(jnp.int32, _L)
    r4b = lane >> 2
    f4 = lane & 3
    sbase = r4b * _L + f4

    def issue_xyz(S, ci):
        off = base + ci * _CHUNK
        pltpu.async_copy(x_hbm.at[pl.ds(off, _CHUNK)], S.x, S.xsem)
        pltpu.async_copy(y_hbm.at[pl.ds(off, _CHUNK)], S.y, S.xsem)
        pltpu.async_copy(z_hbm.at[pl.ds(off, _CHUNK)], S.z, S.xsem)

    def wait_xyz(S):
        for r in (S.x, S.y, S.z):
            pltpu.make_async_copy(x_hbm.at[pl.ds(0, _CHUNK)], r, S.xsem).wait()

    def fire(S):
        for g in range(_CHUNK // _L):
            sl = pl.ds(g * _L, _L)
            x = S.x[sl]
            y = S.y[sl]
            z = S.z[sl]
            px0, pfx = _floorfrac(x, _PLANE_RES)
            py0, pfy = _floorfrac(y, _PLANE_RES)
            pz0, pfz = _floorfrac(z, _PLANE_RES)
            gx0, gfx = _floorfrac(x, _GRID_RES)
            gy0, gfy = _floorfrac(y, _GRID_RES)
            gz0, gfz = _floorfrac(z, _GRID_RES)
            S.fr[pl.ds(0 * _CHUNK + g * _L, _L)] = pfx
            S.fr[pl.ds(1 * _CHUNK + g * _L, _L)] = pfy
            S.fr[pl.ds(2 * _CHUNK + g * _L, _L)] = pfz
            S.fr[pl.ds(3 * _CHUNK + g * _L, _L)] = gfx
            S.fr[pl.ds(4 * _CHUNK + g * _L, _L)] = gfy
            S.fr[pl.ds(5 * _CHUNK + g * _L, _L)] = gfz

            R = _PLANE_RES
            S.pidx[0][sl] = px0 + py0 * R
            S.pidx[1][sl] = py0 + pz0 * R + R * R
            S.pidx[2][sl] = pz0 + px0 * R + 2 * R * R

            G = _GRID_RES
            gb = gx0 + gy0 * G + gz0 * G * G
            qe = gb >> 1
            qo = (gb + 1) >> 1
            for c in range(8):
                dy, dz = (c >> 1) & 1, (c >> 2) & 1
                S.gidx[c][sl] = (qo if (c & 1) else qe) + (
                    dy * (G // 2) + dz * (G * G // 2))
            S.hb[sl] = (gb & 1) << 2

        for c in range(3):
            pltpu.async_copy(plane_hbm.at[S.pidx[c]], S.prow.at[c], S.gsem)
        for c in range(8):
            pltpu.async_copy(grid_hbm.at[S.gidx[c]], S.grow.at[c], S.gsem)

    def drain_gathers(S):
        for c in range(3):
            pltpu.make_async_copy(
                plane_hbm.at[S.pidx[c]], S.prow.at[c], S.gsem).wait()
        for c in range(8):
            pltpu.make_async_copy(
                grid_hbm.at[S.gidx[c]], S.grow.at[c], S.gsem).wait()

    def compute(S, out_v, osem, ci):
        @plsc.parallel_loop(0, _CHUNK // 4)
        def accum4(j):
            r4 = r4b + 4 * j

            def frac(row):
                return plsc.load_gather(S.fr, [row * _CHUNK + r4])

            pfx, pfy, pfz = frac(0), frac(1), frac(2)
            gfx, gfy, gfz = frac(3), frac(4), frac(5)
            one = jnp.float32(1.0)
            four = jnp.int32(4)
            opx, opy, opz = one - pfx, one - pfy, one - pfz
            ogx, ogy, ogz = one - gfx, one - gfy, one - gfz

            h = plsc.load_gather(S.hb, [r4])
            fA = h + f4
            fB = (four - h) + f4

            def row(ref, c, fidx):
                cc = jnp.full((_L,), c, jnp.int32)
                return plsc.load_gather(ref, [cc, r4, fidx])

            pw = (
                opx * opy, pfx * opy, opx * pfy, pfx * pfy,
                opy * opz, pfy * opz, opy * pfz, pfy * pfz,
                opz * opx, pfz * opx, opz * pfx, pfz * pfx,
            )
            for blk in range(3):
                acc = pw[4 * blk] * row(S.prow, blk, f4)
                acc = acc + pw[4 * blk + 1] * row(S.prow, blk, f4 + 4)
                acc = acc + pw[4 * blk + 2] * row(S.prow, blk, f4 + 8)
                acc = acc + pw[4 * blk + 3] * row(S.prow, blk, f4 + 12)
                plsc.store_scatter(out_v, [sbase + (64 * j + 4 * blk)], acc)

            wxy = (ogx * ogy, gfx * ogy, ogx * gfy, gfx * gfy)
            gacc = (wxy[0] * ogz) * row(S.grow, 0, fA)
            for c in range(1, 8):
                w = wxy[c & 3] * (gfz if c >= 4 else ogz)
                gacc = gacc + w * row(S.grow, c, fB if (c & 1) else fA)
            plsc.store_scatter(out_v, [sbase + (64 * j + 12)], gacc)

        off = base + ci * _CHUNK
        pltpu.async_copy(out_v, out_hbm.at[pl.ds(off * _L, _CHUNK * _L)], osem)

    def wait_out(out_v, osem):
        pltpu.make_async_copy(
            out_v, out_hbm.at[pl.ds(0, _CHUNK * _L)], osem).wait()

    for k in range(_NBUF - 1):
        issue_xyz(sets[k], k)
    for k in range(_NBUF - 1):
        wait_xyz(sets[k])
        fire(sets[k])
    issue_xyz(sets[_NBUF - 1], _NBUF - 1)

    def step(i, carry):
        for b in range(_NBUF):
            ci = _NBUF * i + b
            bf = (b + _NBUF - 1) % _NBUF
            wait_xyz(sets[bf])
            fire(sets[bf])
            issue_xyz(sets[b], jnp.minimum(ci + _NBUF, last))
            drain_gathers(sets[b])
            if b < 2:
                @pl.when(i > 0)
                def _():
                    wait_out(outs[b], osems[b])
            else:
                wait_out(outs[b % 2], osems[b % 2])
            compute(sets[b], outs[b % 2], osems[b % 2], ci)
        return carry

    lax.fori_loop(0, nchunk // _NBUF, step, 0)

    for k in range(_NBUF - 1):
        drain_gathers(sets[k])
    wait_xyz(sets[_NBUF - 1])
    wait_out(out0, osem0)
    wait_out(out1, osem1)


def kernel(xyzs, plane_embedding, grid_embedding):
    B = xyzs.shape[0]
    xt = xyzs.T
    x, y, z = xt[0], xt[1], xt[2]
    R = _PLANE_RES
    p4 = plane_embedding.reshape(3, R, R, _FEAT)
    px1 = jnp.roll(p4, -1, axis=2)
    py1 = jnp.roll(p4, -1, axis=1)
    pxy = jnp.roll(px1, -1, axis=1)
    planes = jnp.concatenate([p4, px1, py1, pxy], axis=-1).reshape(
        3 * R * R, 4 * _FEAT)
    grid = grid_embedding.reshape(_GRID_RES ** 3 // 2, 2 * _FEAT)

    mesh = plsc.VectorSubcoreMesh(core_axis_name="c", subcore_axis_name="s")
    run = pl.kernel(
        _body,
        out_type=jax.ShapeDtypeStruct((B * 16,), jnp.float32),
        mesh=mesh,
        compiler_params=pltpu.CompilerParams(
            needs_layout_passes=False, use_tc_tiling_on_sc=False),
        scratch_types=(
            _set_types() * _NBUF
            + [pltpu.VMEM((_CHUNK * _L,), jnp.float32),
               pltpu.SemaphoreType.DMA,
               pltpu.VMEM((_CHUNK * _L,), jnp.float32),
               pltpu.SemaphoreType.DMA]
        ),
    )
    out = run(x, y, z, planes, grid)
    return out.reshape(B, 16)


# speedup vs baseline: 5.1126x; 5.1126x over previous
import jax
import jax.numpy as jnp
from jax import lax
from jax.experimental import pallas as pl
from jax.experimental.pallas import tpu as pltpu
from jax.experimental.pallas import tpu_sc as plsc

_PLANE_RES = 1024
_GRID_RES = 256
_FEAT = 4
_NC = 2
_NS = 16
_NW = _NC * _NS
_L = 16
_CHUNK = 128


def _floorfrac(v, res):
    p = v * jnp.float32(res - 1) + jnp.float32(0.5)
    i = p.astype(jnp.int32)
    f = p - i.astype(jnp.float32)
    ic = jnp.minimum(jnp.maximum(i, 0), res - 2)
    return ic, f


class _Set:
    def __init__(self, s):
        (self.xyz, self.fr, self.hb, self.pidx, self.gidx,
         self.prow, self.grow, self.xsem, self.gsem) = s


_SET_LEN = 9


def _set_types():
    return [
        pltpu.VMEM((3 * _CHUNK,), jnp.float32),
        pltpu.VMEM((6 * _CHUNK,), jnp.float32),
        pltpu.VMEM((4 * _CHUNK,), jnp.int32),
        pltpu.VMEM((12 * _CHUNK,), jnp.int32),
        pltpu.VMEM((8 * _CHUNK,), jnp.int32),
        pltpu.VMEM((12 * _CHUNK, 2 * _FEAT), jnp.float32),
        pltpu.VMEM((8 * _CHUNK, 2 * _FEAT), jnp.float32),
        pltpu.SemaphoreType.DMA,
        pltpu.SemaphoreType.DMA,
    ]


def _body(xyz_hbm, plane_hbm, grid_hbm, out_hbm, *s):
    S0 = _Set(s[0:_SET_LEN])
    S1 = _Set(s[_SET_LEN:2 * _SET_LEN])
    out0, osem0, out1, osem1 = s[2 * _SET_LEN:2 * _SET_LEN + 4]

    wid = lax.axis_index("s") * _NC + lax.axis_index("c")
    npts = xyz_hbm.shape[0] // (3 * _NW)
    nchunk = npts // _CHUNK
    last = nchunk - 1
    base = wid * npts

    lane = lax.iota(jnp.int32, _L)
    r4b = lane >> 2
    f4 = lane & 3
    sbase = r4b * _L + f4

    def issue_xyz(S, ci):
        off = base + ci * _CHUNK
        pltpu.async_copy(
            xyz_hbm.at[pl.ds(off * 3, 3 * _CHUNK)], S.xyz, S.xsem)

    def wait_xyz(S):
        pltpu.make_async_copy(
            xyz_hbm.at[pl.ds(0, 3 * _CHUNK)], S.xyz, S.xsem).wait()

    def fire(S):
        lane3 = lane * 3
        for g in range(_CHUNK // _L):
            sl = pl.ds(g * _L, _L)
            gbase = lane3 + (3 * _L) * g
            x = plsc.load_gather(S.xyz, [gbase])
            y = plsc.load_gather(S.xyz, [gbase + 1])
            z = plsc.load_gather(S.xyz, [gbase + 2])
            px0, pfx = _floorfrac(x, _PLANE_RES)
            py0, pfy = _floorfrac(y, _PLANE_RES)
            pz0, pfz = _floorfrac(z, _PLANE_RES)
            gx0, gfx = _floorfrac(x, _GRID_RES)
            gy0, gfy = _floorfrac(y, _GRID_RES)
            gz0, gfz = _floorfrac(z, _GRID_RES)
            S.fr[pl.ds(0 * _CHUNK + g * _L, _L)] = pfx
            S.fr[pl.ds(1 * _CHUNK + g * _L, _L)] = pfy
            S.fr[pl.ds(2 * _CHUNK + g * _L, _L)] = pfz
            S.fr[pl.ds(3 * _CHUNK + g * _L, _L)] = gfx
            S.fr[pl.ds(4 * _CHUNK + g * _L, _L)] = gfy
            S.fr[pl.ds(5 * _CHUNK + g * _L, _L)] = gfz

            R = _PLANE_RES
            b0 = px0 + py0 * R
            b1 = py0 + pz0 * R + R * R
            b2 = pz0 + px0 * R + 2 * R * R
            for pi, b in enumerate((b0, b1, b2)):
                qe = b >> 1
                qo = (b + 1) >> 1
                S.pidx[pl.ds((4 * pi + 0) * _CHUNK + g * _L, _L)] = qe
                S.pidx[pl.ds((4 * pi + 1) * _CHUNK + g * _L, _L)] = qo
                S.pidx[pl.ds((4 * pi + 2) * _CHUNK + g * _L, _L)] = qe + R // 2
                S.pidx[pl.ds((4 * pi + 3) * _CHUNK + g * _L, _L)] = qo + R // 2
                S.hb[pl.ds(pi * _CHUNK + g * _L, _L)] = (b & 1) << 2

            G = _GRID_RES
            gb = gx0 + gy0 * G + gz0 * G * G
            qe = gb >> 1
            qo = (gb + 1) >> 1
            for c in range(8):
                dy, dz = (c >> 1) & 1, (c >> 2) & 1
                S.gidx[pl.ds(c * _CHUNK + g * _L, _L)] = (
                    qo if (c & 1) else qe) + (
                    dy * (G // 2) + dz * (G * G // 2))
            S.hb[pl.ds(3 * _CHUNK + g * _L, _L)] = (gb & 1) << 2

        pltpu.async_copy(plane_hbm.at[S.pidx], S.prow, S.gsem)
        pltpu.async_copy(grid_hbm.at[S.gidx], S.grow, S.gsem)

    def drain_gathers(S):
        pltpu.make_async_copy(plane_hbm.at[S.pidx], S.prow, S.gsem).wait()
        pltpu.make_async_copy(grid_hbm.at[S.gidx], S.grow, S.gsem).wait()

    def compute(S, out_v, osem, ci):
        @plsc.parallel_loop(0, _CHUNK // 4)
        def accum4(j):
            r4 = r4b + 4 * j

            def frac(row):
                return plsc.load_gather(S.fr, [row * _CHUNK + r4])

            pfx, pfy, pfz = frac(0), frac(1), frac(2)
            gfx, gfy, gfz = frac(3), frac(4), frac(5)
            one = jnp.float32(1.0)
            four = jnp.int32(4)
            opx, opy, opz = one - pfx, one - pfy, one - pfz
            ogx, ogy, ogz = one - gfx, one - gfy, one - gfz

            hs = [plsc.load_gather(S.hb, [k * _CHUNK + r4]) for k in range(4)]
            fA = [h + f4 for h in hs]
            fB = [(four - h) + f4 for h in hs]

            def row(ref, c, fidx):
                return plsc.load_gather(ref, [r4 + c * _CHUNK, fidx])

            pw = (
                opx * opy, pfx * opy, opx * pfy, pfx * pfy,
                opy * opz, pfy * opz, opy * pfz, pfy * pfz,
                opz * opx, pfz * opx, opz * pfx, pfz * pfx,
            )
            for blk in range(3):
                acc = pw[4 * blk] * row(S.prow, 4 * blk, fA[blk])
                acc = acc + pw[4 * blk + 1] * row(S.prow, 4 * blk + 1, fB[blk])
                acc = acc + pw[4 * blk + 2] * row(S.prow, 4 * blk + 2, fA[blk])
                acc = acc + pw[4 * blk + 3] * row(S.prow, 4 * blk + 3, fB[blk])
                plsc.store_scatter(out_v, [sbase + (64 * j + 4 * blk)], acc)

            wxy = (ogx * ogy, gfx * ogy, ogx * gfy, gfx * gfy)
            gacc = (wxy[0] * ogz) * row(S.grow, 0, fA[3])
            for c in range(1, 8):
                w = wxy[c & 3] * (gfz if c >= 4 else ogz)
                gacc = gacc + w * row(S.grow, c, fB[3] if (c & 1) else fA[3])
            plsc.store_scatter(out_v, [sbase + (64 * j + 12)], gacc)

        off = base + ci * _CHUNK
        pltpu.async_copy(out_v, out_hbm.at[pl.ds(off * _L, _CHUNK * _L)], osem)

    def wait_out(out_v, osem):
        pltpu.make_async_copy(
            out_v, out_hbm.at[pl.ds(0, _CHUNK * _L)], osem).wait()

    issue_xyz(S0, 0)
    wait_xyz(S0)
    fire(S0)
    issue_xyz(S1, 1)

    def step(i, carry):
        c0 = 2 * i
        c1 = c0 + 1
        wait_xyz(S1)
        fire(S1)
        issue_xyz(S0, jnp.minimum(c0 + 2, last))
        drain_gathers(S0)

        @pl.when(i > 0)
        def _():
            wait_out(out0, osem0)

        compute(S0, out0, osem0, c0)
        wait_xyz(S0)
        fire(S0)
        issue_xyz(S1, jnp.minimum(c1 + 2, last))
        drain_gathers(S1)

        @pl.when(i > 0)
        def _():
            wait_out(out1, osem1)

        compute(S1, out1, osem1, c1)
        return carry

    lax.fori_loop(0, nchunk // 2, step, 0)

    drain_gathers(S0)
    wait_xyz(S1)
    wait_out(out0, osem0)
    wait_out(out1, osem1)


def kernel(xyzs, plane_embedding, grid_embedding):
    B = xyzs.shape[0]
    xyz_flat = xyzs.reshape(3 * B)
    planes = plane_embedding.reshape(3 * _PLANE_RES * _PLANE_RES // 2, 2 * _FEAT)
    grid = grid_embedding.reshape(_GRID_RES ** 3 // 2, 2 * _FEAT)

    mesh = plsc.VectorSubcoreMesh(core_axis_name="c", subcore_axis_name="s")
    run = pl.kernel(
        _body,
        out_type=jax.ShapeDtypeStruct((B * 16,), jnp.float32),
        mesh=mesh,
        compiler_params=pltpu.CompilerParams(
            needs_layout_passes=False, use_tc_tiling_on_sc=False),
        scratch_types=(
            _set_types() + _set_types()
            + [pltpu.VMEM((_CHUNK * _L,), jnp.float32),
               pltpu.SemaphoreType.DMA,
               pltpu.VMEM((_CHUNK * _L,), jnp.float32),
               pltpu.SemaphoreType.DMA]
        ),
    )
    out = run(xyz_flat, planes, grid)
    return out.reshape(B, 16)


# speedup vs baseline: 10.7344x; 2.0996x over previous
import jax
import jax.numpy as jnp
from jax import lax
from jax.experimental import pallas as pl
from jax.experimental.pallas import tpu as pltpu
from jax.experimental.pallas import tpu_sc as plsc

_PLANE_RES = 1024
_GRID_RES = 256
_FEAT = 4
_NC = 2
_NS = 16
_NW = _NC * _NS
_L = 16
_CHUNK = 128


def _floorfrac(v, res):
    p = v * jnp.float32(res - 1) + jnp.float32(0.5)
    i = p.astype(jnp.int32)
    f = p - i.astype(jnp.float32)
    ic = jnp.minimum(jnp.maximum(i, 0), res - 2)
    return ic, f


class _Set:
    def __init__(self, s):
        (self.x, self.y, self.z, self.fr, self.hb) = s[0:5]
        self.pidx = s[5:17]
        self.gidx = s[17:25]
        (self.prow, self.grow, self.xsem, self.gsem) = s[25:29]


_SET_LEN = 29


def _set_types():
    return (
        [pltpu.VMEM((_CHUNK,), jnp.float32)] * 3
        + [pltpu.VMEM((6 * _CHUNK,), jnp.float32)]
        + [pltpu.VMEM((4 * _CHUNK,), jnp.int32)]
        + [pltpu.VMEM((_CHUNK,), jnp.int32)] * 20
        + [pltpu.VMEM((12, _CHUNK, 2 * _FEAT), jnp.float32),
           pltpu.VMEM((8, _CHUNK, 2 * _FEAT), jnp.float32),
           pltpu.SemaphoreType.DMA,
           pltpu.SemaphoreType.DMA]
    )


def _body(x_hbm, y_hbm, z_hbm, plane_hbm, grid_hbm, out_hbm, *s):
    S0 = _Set(s[0:_SET_LEN])
    S1 = _Set(s[_SET_LEN:2 * _SET_LEN])
    out0, osem0, out1, osem1 = s[2 * _SET_LEN:2 * _SET_LEN + 4]

    wid = lax.axis_index("s") * _NC + lax.axis_index("c")
    npts = x_hbm.shape[0] // _NW
    nchunk = npts // _CHUNK
    last = nchunk - 1
    base = wid * npts

    lane = lax.iota(jnp.int32, _L)
    r4b = lane >> 2
    f4 = lane & 3
    sbase = r4b * _L + f4

    def issue_xyz(S, ci):
        off = base + ci * _CHUNK
        pltpu.async_copy(x_hbm.at[pl.ds(off, _CHUNK)], S.x, S.xsem)
        pltpu.async_copy(y_hbm.at[pl.ds(off, _CHUNK)], S.y, S.xsem)
        pltpu.async_copy(z_hbm.at[pl.ds(off, _CHUNK)], S.z, S.xsem)

    def wait_xyz(S):
        for r in (S.x, S.y, S.z):
            pltpu.make_async_copy(x_hbm.at[pl.ds(0, _CHUNK)], r, S.xsem).wait()

    def fire(S):
        for g in range(_CHUNK // _L):
            sl = pl.ds(g * _L, _L)
            x = S.x[sl]
            y = S.y[sl]
            z = S.z[sl]
            px0, pfx = _floorfrac(x, _PLANE_RES)
            py0, pfy = _floorfrac(y, _PLANE_RES)
            pz0, pfz = _floorfrac(z, _PLANE_RES)
            gx0, gfx = _floorfrac(x, _GRID_RES)
            gy0, gfy = _floorfrac(y, _GRID_RES)
            gz0, gfz = _floorfrac(z, _GRID_RES)
            S.fr[pl.ds(0 * _CHUNK + g * _L, _L)] = pfx
            S.fr[pl.ds(1 * _CHUNK + g * _L, _L)] = pfy
            S.fr[pl.ds(2 * _CHUNK + g * _L, _L)] = pfz
            S.fr[pl.ds(3 * _CHUNK + g * _L, _L)] = gfx
            S.fr[pl.ds(4 * _CHUNK + g * _L, _L)] = gfy
            S.fr[pl.ds(5 * _CHUNK + g * _L, _L)] = gfz

            R = _PLANE_RES
            b0 = px0 + py0 * R
            b1 = py0 + pz0 * R + R * R
            b2 = pz0 + px0 * R + 2 * R * R
            for pi, b in enumerate((b0, b1, b2)):
                S.pidx[4 * pi + 0][sl] = b >> 1
                S.pidx[4 * pi + 1][sl] = (b + 1) >> 1
                S.pidx[4 * pi + 2][sl] = (b + R) >> 1
                S.pidx[4 * pi + 3][sl] = (b + R + 1) >> 1
                S.hb[pl.ds(pi * _CHUNK + g * _L, _L)] = (b & 1) << 2

            G = _GRID_RES
            gb = gx0 + gy0 * G + gz0 * G * G
            for c in range(8):
                dx, dy, dz = c & 1, (c >> 1) & 1, (c >> 2) & 1
                S.gidx[c][sl] = (gb + (dx + dy * G + dz * G * G)) >> 1
            S.hb[pl.ds(3 * _CHUNK + g * _L, _L)] = (gb & 1) << 2

        for c in range(12):
            for hh in range(2):
                pltpu.async_copy(
                    plane_hbm.at[S.pidx[c].at[pl.ds(hh * 64, 64)]],
                    S.prow.at[c].at[pl.ds(hh * 64, 64)], S.gsem)
        for c in range(8):
            for hh in range(2):
                pltpu.async_copy(
                    grid_hbm.at[S.gidx[c].at[pl.ds(hh * 64, 64)]],
                    S.grow.at[c].at[pl.ds(hh * 64, 64)], S.gsem)

    def drain_gathers(S):
        for c in range(12):
            for hh in range(2):
                pltpu.make_async_copy(
                    plane_hbm.at[S.pidx[c].at[pl.ds(hh * 64, 64)]],
                    S.prow.at[c].at[pl.ds(hh * 64, 64)], S.gsem).wait()
        for c in range(8):
            for hh in range(2):
                pltpu.make_async_copy(
                    grid_hbm.at[S.gidx[c].at[pl.ds(hh * 64, 64)]],
                    S.grow.at[c].at[pl.ds(hh * 64, 64)], S.gsem).wait()

    def compute(S, out_v, osem, ci):
        @plsc.parallel_loop(0, _CHUNK // 4)
        def accum4(j):
            r4 = r4b + 4 * j

            def frac(row):
                return plsc.load_gather(S.fr, [row * _CHUNK + r4])

            pfx, pfy, pfz = frac(0), frac(1), frac(2)
            gfx, gfy, gfz = frac(3), frac(4), frac(5)
            one = jnp.float32(1.0)
            four = jnp.int32(4)
            opx, opy, opz = one - pfx, one - pfy, one - pfz
            ogx, ogy, ogz = one - gfx, one - gfy, one - gfz

            hs = [plsc.load_gather(S.hb, [k * _CHUNK + r4]) for k in range(4)]
            fA = [h + f4 for h in hs]
            fB = [(four - h) + f4 for h in hs]

            def row(ref, c, fidx):
                cc = jnp.full((_L,), c, jnp.int32)
                return plsc.load_gather(ref, [cc, r4, fidx])

            pw = (
                opx * opy, pfx * opy, opx * pfy, pfx * pfy,
                opy * opz, pfy * opz, opy * pfz, pfy * pfz,
                opz * opx, pfz * opx, opz * pfx, pfz * pfx,
            )
            for blk in range(3):
                acc = pw[4 * blk] * row(S.prow, 4 * blk, fA[blk])
                acc = acc + pw[4 * blk + 1] * row(S.prow, 4 * blk + 1, fB[blk])
                acc = acc + pw[4 * blk + 2] * row(S.prow, 4 * blk + 2, fA[blk])
                acc = acc + pw[4 * blk + 3] * row(S.prow, 4 * blk + 3, fB[blk])
                plsc.store_scatter(out_v, [sbase + (64 * j + 4 * blk)], acc)

            wxy = (ogx * ogy, gfx * ogy, ogx * gfy, gfx * gfy)
            gacc = (wxy[0] * ogz) * row(S.grow, 0, fA[3])
            for c in range(1, 8):
                w = wxy[c & 3] * (gfz if c >= 4 else ogz)
                gacc = gacc + w * row(S.grow, c, fB[3] if (c & 1) else fA[3])
            plsc.store_scatter(out_v, [sbase + (64 * j + 12)], gacc)

        off = base + ci * _CHUNK
        pltpu.async_copy(out_v, out_hbm.at[pl.ds(off * _L, _CHUNK * _L)], osem)

    def wait_out(out_v, osem):
        pltpu.make_async_copy(
            out_v, out_hbm.at[pl.ds(0, _CHUNK * _L)], osem).wait()

    issue_xyz(S0, 0)
    wait_xyz(S0)
    fire(S0)
    issue_xyz(S1, 1)

    def step(i, carry):
        c0 = 2 * i
        c1 = c0 + 1
        wait_xyz(S1)
        fire(S1)
        issue_xyz(S0, jnp.minimum(c0 + 2, last))
        drain_gathers(S0)

        @pl.when(i > 0)
        def _():
            wait_out(out0, osem0)

        compute(S0, out0, osem0, c0)
        wait_xyz(S0)
        fire(S0)
        issue_xyz(S1, jnp.minimum(c1 + 2, last))
        drain_gathers(S1)

        @pl.when(i > 0)
        def _():
            wait_out(out1, osem1)

        compute(S1, out1, osem1, c1)
        return carry

    lax.fori_loop(0, nchunk // 2, step, 0)

    drain_gathers(S0)
    wait_xyz(S1)
    wait_out(out0, osem0)
    wait_out(out1, osem1)


def kernel(xyzs, plane_embedding, grid_embedding):
    B = xyzs.shape[0]
    xt = xyzs.T
    x, y, z = xt[0], xt[1], xt[2]
    planes = plane_embedding.reshape(3 * _PLANE_RES * _PLANE_RES // 2, 2 * _FEAT)
    grid = grid_embedding.reshape(_GRID_RES ** 3 // 2, 2 * _FEAT)

    mesh = plsc.VectorSubcoreMesh(core_axis_name="c", subcore_axis_name="s")
    run = pl.kernel(
        _body,
        out_type=jax.ShapeDtypeStruct((B * 16,), jnp.float32),
        mesh=mesh,
        compiler_params=pltpu.CompilerParams(
            needs_layout_passes=False, use_tc_tiling_on_sc=False),
        scratch_types=(
            _set_types() + _set_types()
            + [pltpu.VMEM((_CHUNK * _L,), jnp.float32),
               pltpu.SemaphoreType.DMA,
               pltpu.VMEM((_CHUNK * _L,), jnp.float32),
               pltpu.SemaphoreType.DMA]
        ),
    )
    out = run(x, y, z, planes, grid)
    return out.reshape(B, 16)
